# baseline (speedup 1.0000x reference)
import functools

import jax
import jax.numpy as jnp
from jax import lax
from jax.experimental import pallas as pl
from jax.experimental.pallas import tpu as pltpu
from jax.experimental.pallas import tpu_sc as plsc

N = 50000
E = 800000
D = 64
Q = 100000

NC = 2
NS = 16
LANES = 16

EE = 2 * E
SUB = 128
KSUB = 4
CHUNK = SUB * KSUB
EPW_RAW = -(-EE // NS)
EPW = -(-EPW_RAW // CHUNK) * CHUNK
EEP = EPW * NS
NCHUNKS = EPW // CHUNK

NPAD = 50048
ROWS_PER_TILE = NPAD // NS

QPW = -(-Q // (NS * CHUNK)) * CHUNK
QP = QPW * NS


def _mesh():
  return plsc.VectorSubcoreMesh(core_axis_name="c", subcore_axis_name="s")


_SC_PARAMS = pltpu.CompilerParams(use_tc_tiling_on_sc=False)


def _proj_body(emb_ref, w_ref, out_ref):
  out_ref[...] = jnp.dot(emb_ref[...], w_ref[...],
                         preferred_element_type=jnp.float32)


def _proj(emb, wcat):
  bn = 1000
  return pl.pallas_call(
      _proj_body,
      grid=(N // bn,),
      in_specs=[
          pl.BlockSpec((bn, D), lambda i: (i, 0)),
          pl.BlockSpec((D, 4 * D), lambda i: (0, 0)),
      ],
      out_specs=pl.BlockSpec((bn, 4 * D), lambda i: (i, 0)),
      out_shape=jax.ShapeDtypeStruct((N, 4 * D), jnp.float32),
  )(emb, wcat)


def _scatter_body(hsv, gidx_all, dst_all, val_all, hn_out,
                  idxb, dstb, valb, dstu, rows, agg_sh,
                  semS0, semS1, semG, semC):
  c = lax.axis_index("c")
  s = lax.axis_index("s")
  semS = (semS0, semS1)

  def _addc(b):
    for j in range(KSUB):
      def body(g, _, j=j):
        idxb[b, j, pl.ds(g * LANES, LANES)] = (
            idxb[b, j, pl.ds(g * LANES, LANES)] + c)
        return _

      lax.fori_loop(0, SUB // LANES, body, None)

  def _scale(b, j):
    def body(g, _):
      v16 = valb[b, j, pl.ds(g * LANES, LANES)]
      e0 = g * LANES
      for t in range(LANES):
        sc = v16[t]
        rows[j, e0 + t, pl.ds(0, LANES)] = (
            rows[j, e0 + t, pl.ds(0, LANES)] * sc)
        rows[j, e0 + t, pl.ds(LANES, LANES)] = (
            rows[j, e0 + t, pl.ds(LANES, LANES)] * sc)
      return _

    lax.fori_loop(0, SUB // LANES, body, None)

  def _dstu_copy(b, j):
    def body(g, _):
      dstu[j, pl.ds(g * LANES, LANES)] = dstb[b, j, pl.ds(g * LANES, LANES)]
      return _

    lax.fori_loop(0, SUB // LANES, body, None)

  def _stage(row, b, sem):
    pltpu.async_copy(gidx_all.at[row], idxb.at[b], sem)
    pltpu.async_copy(dst_all.at[row], dstb.at[b], sem)
    pltpu.async_copy(val_all.at[row], valb.at[b], sem)

  def _stage_wait(row, b, sem):
    pltpu.make_async_copy(gidx_all.at[row], idxb.at[b], sem).wait()
    pltpu.make_async_copy(dst_all.at[row], dstb.at[b], sem).wait()
    pltpu.make_async_copy(val_all.at[row], valb.at[b], sem).wait()

  zr0 = s * ROWS_PER_TILE
  nfull = ROWS_PER_TILE // SUB
  ztail = ROWS_PER_TILE - nfull * SUB
  iota16 = lax.iota(jnp.int32, LANES)

  def init_round(m, _):
    node0 = zr0 + m * SUB
    for g in range(SUB // LANES):
      node16 = jnp.minimum(node0 + g * LANES + iota16, N - 1)
      dstu[0, pl.ds(g * LANES, LANES)] = node16 * 8 + 4 + c
    pltpu.async_copy(hsv.at[dstu.at[0]], rows.at[0], semG)
    pltpu.make_async_copy(hsv.at[dstu.at[0]], rows.at[0], semG).wait()
    pltpu.sync_copy(rows.at[0], agg_sh.at[pl.ds(node0, SUB)])
    return _

  lax.fori_loop(0, nfull, init_round, None)
  node0t = zr0 + nfull * SUB
  for g in range(SUB // LANES):
    node16 = jnp.minimum(node0t + g * LANES + iota16, N - 1)
    dstu[0, pl.ds(g * LANES, LANES)] = node16 * 8 + 4 + c
  pltpu.async_copy(hsv.at[dstu.at[0]], rows.at[0], semG)
  pltpu.make_async_copy(hsv.at[dstu.at[0]], rows.at[0], semG).wait()
  pltpu.sync_copy(rows.at[0, pl.ds(0, ztail)],
                  agg_sh.at[pl.ds(node0t, ztail)])
  plsc.subcore_barrier()

  chunk0 = s * NCHUNKS

  pltpu.sync_copy(gidx_all.at[chunk0], idxb.at[0])
  pltpu.sync_copy(dst_all.at[chunk0], dstb.at[0])
  pltpu.sync_copy(val_all.at[chunk0], valb.at[0])
  pltpu.sync_copy(gidx_all.at[chunk0 + 1], idxb.at[1])
  pltpu.sync_copy(dst_all.at[chunk0 + 1], dstb.at[1])
  pltpu.sync_copy(val_all.at[chunk0 + 1], valb.at[1])
  _addc(0)
  _addc(1)
  pltpu.async_copy(hsv.at[idxb.at[0, 0]], rows.at[0], semG)
  pltpu.async_copy(hsv.at[idxb.at[0, 1]], rows.at[1], semG)

  def pair_body(p, _):
    for sb in range(2):
      lc = 2 * p + sb
      r = chunk0 + lc

      for j in range(KSUB):
        pltpu.make_async_copy(hsv.at[idxb.at[sb, j]], rows.at[j],
                              semG).wait()
        j2 = (j - 2) % KSUB

        @pl.when(4 * lc + j >= 2)
        def _():
          pltpu.make_async_copy(rows.at[j2], agg_sh.at[dstu.at[j2]],
                                semC).wait()

        if j == 2:
          @pl.when(jnp.logical_and(lc >= 1, lc + 1 < NCHUNKS))
          def _():
            _stage_wait(r + 1, 1 - sb, semS[1 - sb])
            _addc(1 - sb)

        if j < 2:
          pltpu.async_copy(hsv.at[idxb.at[sb, j + 2]], rows.at[j + 2], semG)
        else:
          @pl.when(lc + 1 < NCHUNKS)
          def _():
            pltpu.async_copy(hsv.at[idxb.at[1 - sb, j - 2]], rows.at[j - 2],
                             semG)

        _dstu_copy(sb, j)
        _scale(sb, j)
        pltpu.async_copy(rows.at[j], agg_sh.at[dstu.at[j]], semC, add=True)

      @pl.when(lc + 2 < NCHUNKS)
      def _():
        _stage(r + 2, sb, semS[sb])
    return _

  lax.fori_loop(0, NCHUNKS // 2, pair_body, None)
  for j2 in (2, 3):
    pltpu.make_async_copy(rows.at[j2], agg_sh.at[dstu.at[j2]], semC).wait()
  plsc.subcore_barrier()

  def wo_round(m, nr):
    node0 = zr0 + m * SUB
    pltpu.sync_copy(agg_sh.at[pl.ds(node0, nr)], rows.at[0, pl.ds(0, nr)])

    def relu_body(i, _):
      rows[0, i, pl.ds(0, LANES)] = jnp.maximum(
          rows[0, i, pl.ds(0, LANES)], 0.0)
      rows[0, i, pl.ds(LANES, LANES)] = jnp.maximum(
          rows[0, i, pl.ds(LANES, LANES)], 0.0)
      return _

    lax.fori_loop(0, nr, relu_body, None)
    pltpu.sync_copy(rows.at[0, pl.ds(0, nr)],
                    hn_out.at[pl.ds(c * NPAD + node0, nr)])

  def wo_body(m, _):
    wo_round(m, SUB)
    return _

  lax.fori_loop(0, nfull, wo_body, None)
  wo_round(nfull, ztail)


def _scatter(hsv, gidx_all, dst_all, val_all):
  k = pl.kernel(
      _scatter_body,
      out_type=jax.ShapeDtypeStruct((NC * NPAD, D // 2), jnp.float32),
      mesh=_mesh(),
      scratch_types=[
          pltpu.VMEM((2, KSUB, SUB), jnp.int32),
          pltpu.VMEM((2, KSUB, SUB), jnp.int32),
          pltpu.VMEM((2, KSUB, SUB), jnp.float32),
          pltpu.VMEM((KSUB, SUB), jnp.int32),
          pltpu.VMEM((KSUB, SUB, D // 2), jnp.float32),
          pltpu.VMEM_SHARED((NPAD, D // 2), jnp.float32),
          pltpu.SemaphoreType.DMA,
          pltpu.SemaphoreType.DMA,
          pltpu.SemaphoreType.DMA,
          pltpu.SemaphoreType.DMA,
      ],
      compiler_params=_SC_PARAMS,
  )
  return k(hsv, gidx_all, dst_all, val_all)


def _qgather_body(hn, qidx, hp_out, idxb, idxb2, rowb, sem):
  c = lax.axis_index("c")
  s = lax.axis_index("s")
  nchunks = QPW // CHUNK

  def body(i, _):
    qc = c * (QP // CHUNK) + s * nchunks + i
    r = (s * nchunks + i) * SUB * KSUB
    pltpu.sync_copy(qidx.at[qc], idxb)
    for j in range(KSUB):
      def addp(g, _, j=j):
        idxb2[j, pl.ds(g * LANES, LANES)] = (
            idxb[j, pl.ds(g * LANES, LANES)] + NPAD)
        return _

      lax.fori_loop(0, SUB // LANES, addp, None)
    for j in range(KSUB):
      pltpu.async_copy(hn.at[idxb.at[j]],
                       rowb.at[0, pl.ds(j * SUB, SUB)], sem)
      pltpu.async_copy(hn.at[idxb2.at[j]],
                       rowb.at[1, pl.ds(j * SUB, SUB)], sem)
    for j in range(KSUB):
      pltpu.make_async_copy(hn.at[idxb.at[j]],
                            rowb.at[0, pl.ds(j * SUB, SUB)], sem).wait()
      pltpu.make_async_copy(hn.at[idxb2.at[j]],
                            rowb.at[1, pl.ds(j * SUB, SUB)], sem).wait()
    pltpu.sync_copy(rowb.at[0], hp_out.at[pl.ds(2 * c * QP + r, CHUNK)])
    pltpu.sync_copy(rowb.at[1],
                    hp_out.at[pl.ds((2 * c + 1) * QP + r, CHUNK)])
    return _

  lax.fori_loop(0, nchunks, body, None)


def _qgather(hn, qidx):
  k = pl.kernel(
      _qgather_body,
      out_type=jax.ShapeDtypeStruct((4 * QP, D // 2), jnp.float32),
      mesh=_mesh(),
      scratch_types=[
          pltpu.VMEM((KSUB, SUB), jnp.int32),
          pltpu.VMEM((KSUB, SUB), jnp.int32),
          pltpu.VMEM((2, CHUNK, D // 2), jnp.float32),
          pltpu.SemaphoreType.DMA,
      ],
      compiler_params=_SC_PARAMS,
  )
  return k(hn, qidx)


def _head_body(u0_ref, u1_ref, v0_ref, v1_ref, bu0_ref, bu1_ref, bv0_ref,
               bv1_ref, b1_ref, w2_ref, b2_ref, out_ref):
  z = jnp.maximum(
      jnp.dot(u0_ref[...], bu0_ref[...], preferred_element_type=jnp.float32)
      + jnp.dot(u1_ref[...], bu1_ref[...], preferred_element_type=jnp.float32)
      + jnp.dot(v0_ref[...], bv0_ref[...], preferred_element_type=jnp.float32)
      + jnp.dot(v1_ref[...], bv1_ref[...], preferred_element_type=jnp.float32)
      + b1_ref[...], 0.0)
  l = jnp.dot(z, w2_ref[...], preferred_element_type=jnp.float32) + b2_ref[...]
  parts = []
  for g in range(4):
    lg = l[:, 4 * g:4 * g + 4]
    m = jnp.max(lg, axis=-1, keepdims=True)
    e = jnp.exp(lg - m)
    parts.append(e / jnp.sum(e, axis=-1, keepdims=True))
  out_ref[...] = jnp.concatenate(parts, axis=-1)


BQ4 = 416
Q4 = Q // 4


def _head(hp4, bu0, bu1, bv0, bv1, b1, w2, b2):
  sec = QP // 4 // BQ4
  return pl.pallas_call(
      _head_body,
      grid=(-(-Q4 // BQ4),),
      in_specs=[
          pl.BlockSpec((BQ4, 2 * D), lambda i: (i, 0)),
          pl.BlockSpec((BQ4, 2 * D), lambda i: (sec + i, 0)),
          pl.BlockSpec((BQ4, 2 * D), lambda i: (2 * sec + i, 0)),
          pl.BlockSpec((BQ4, 2 * D), lambda i: (3 * sec + i, 0)),
          pl.BlockSpec((2 * D, 4 * D), lambda i: (0, 0)),
          pl.BlockSpec((2 * D, 4 * D), lambda i: (0, 0)),
          pl.BlockSpec((2 * D, 4 * D), lambda i: (0, 0)),
          pl.BlockSpec((2 * D, 4 * D), lambda i: (0, 0)),
          pl.BlockSpec((1, 4 * D), lambda i: (0, 0)),
          pl.BlockSpec((4 * D, 16), lambda i: (0, 0)),
          pl.BlockSpec((1, 16), lambda i: (0, 0)),
      ],
      out_specs=pl.BlockSpec((BQ4, 16), lambda i: (i, 0)),
      out_shape=jax.ShapeDtypeStruct((Q4, 16), jnp.float32),
  )(hp4, hp4, hp4, hp4, bu0, bu1, bv0, bv1, b1, w2, b2)


def kernel(node_ids, omni_idx, omni_val, string_idx, string_val, query_edges,
           emb_table, W_self, W_omni, W_string, Wc1, bc1, Wc2, bc2):
  wcat = jnp.concatenate([W_omni, W_string, W_self, W_self], axis=1)
  hs = _proj(emb_table, wcat)
  hsv = hs.reshape(8 * N, D // 2)

  pad = EEP - EE
  par = jnp.arange(pad, dtype=jnp.int32)
  gidx_all = jnp.concatenate([
      omni_idx[:, 1] * 8,
      string_idx[:, 1] * 8 + 2,
      (par % N) * 8,
  ]).reshape(EEP // CHUNK, KSUB, SUB)
  dst_all = jnp.concatenate(
      [omni_idx[:, 0], string_idx[:, 0],
       par % N]).reshape(EEP // CHUNK, KSUB, SUB)
  val_all = jnp.concatenate([
      omni_val, string_val, jnp.zeros((pad,), jnp.float32)
  ]).reshape(EEP // CHUNK, KSUB, SUB)

  hn = _scatter(hsv, gidx_all, dst_all, val_all)

  qpad = QP - Q
  qpar = jnp.arange(qpad, dtype=jnp.int32) % N
  qidx = jnp.concatenate([
      query_edges[:, 0], qpar,
      query_edges[:, 1], qpar,
  ]).reshape(NC * QP // CHUNK, KSUB, SUB)

  hp = _qgather(hn, qidx)
  hp4 = hp.reshape(QP, 2 * D)

  eye4 = jnp.eye(4, dtype=jnp.float32)
  w1a, w1b = Wc1[:D], Wc1[D:]
  bu0 = jnp.kron(eye4, w1a[:D // 2])
  bu1 = jnp.kron(eye4, w1a[D // 2:])
  bv0 = jnp.kron(eye4, w1b[:D // 2])
  bv1 = jnp.kron(eye4, w1b[D // 2:])
  b1 = jnp.tile(bc1, 4).reshape(1, 4 * D)
  w2 = jnp.kron(eye4, Wc2)
  b2 = jnp.tile(bc2, 4).reshape(1, 16)

  probs4 = _head(hp4, bu0, bu1, bv0, bv1, b1, w2, b2)
  return probs4.reshape(Q, 4)


# speedup vs baseline: 1.0277x; 1.0277x over previous
import functools

import jax
import jax.numpy as jnp
from jax import lax
from jax.experimental import pallas as pl
from jax.experimental.pallas import tpu as pltpu
from jax.experimental.pallas import tpu_sc as plsc

N = 50000
E = 800000
D = 64
Q = 100000

NC = 2
NS = 16
LANES = 16

EE = 2 * E
SUB = 128
KSUB = 4
CHUNK = SUB * KSUB
EPW_RAW = -(-EE // NS)
EPW = -(-EPW_RAW // CHUNK) * CHUNK
EEP = EPW * NS
NCHUNKS = EPW // CHUNK

NPAD = 50048
ROWS_PER_TILE = NPAD // NS

QPW = -(-Q // (NS * CHUNK)) * CHUNK
QP = QPW * NS


def _mesh():
  return plsc.VectorSubcoreMesh(core_axis_name="c", subcore_axis_name="s")


_SC_PARAMS = pltpu.CompilerParams(use_tc_tiling_on_sc=False)


def _proj_body(emb_ref, w1_ref, w2_ref, out1_ref, out2_ref):
  out1_ref[...] = jnp.dot(emb_ref[...], w1_ref[...],
                          preferred_element_type=jnp.float32)
  out2_ref[...] = jnp.dot(emb_ref[...], w2_ref[...],
                          preferred_element_type=jnp.float32)


def _proj(emb, wcat1, wcat2):
  bn = 1000
  return pl.pallas_call(
      _proj_body,
      grid=(N // bn,),
      in_specs=[
          pl.BlockSpec((bn, D), lambda i: (i, 0)),
          pl.BlockSpec((D, 2 * D), lambda i: (0, 0)),
          pl.BlockSpec((D, 2 * D), lambda i: (0, 0)),
      ],
      out_specs=[
          pl.BlockSpec((bn, 2 * D), lambda i: (i, 0)),
          pl.BlockSpec((bn, 2 * D), lambda i: (i, 0)),
      ],
      out_shape=[
          jax.ShapeDtypeStruct((N, 2 * D), jnp.float32),
          jax.ShapeDtypeStruct((N, 2 * D), jnp.float32),
      ],
  )(emb, wcat1, wcat2)


QW = 2 * QP // NS
QUNITS = QW // SUB
QCH = QW // CHUNK


def _scatter_body(hsv1, hsv2, gidx_all, dst_all, val_all, qidx, hp_out,
                  idxb, dstb, valb, dstu, rows, agg_sh,
                  semS0, semS1, semG, semC):
  c = lax.axis_index("c")
  s = lax.axis_index("s")
  semS = (semS0, semS1)

  def _addc(b):
    for j in range(KSUB):
      def body(g, _, j=j):
        idxb[b, j, pl.ds(g * LANES, LANES)] = (
            idxb[b, j, pl.ds(g * LANES, LANES)] + c)
        return _

      lax.fori_loop(0, SUB // LANES, body, None)

  def _scale(b, j):
    def body(g, _):
      v16 = valb[b, j, pl.ds(g * LANES, LANES)]
      e0 = g * LANES
      for t in range(LANES):
        sc = v16[t]
        rows[j, e0 + t, pl.ds(0, LANES)] = (
            rows[j, e0 + t, pl.ds(0, LANES)] * sc)
        rows[j, e0 + t, pl.ds(LANES, LANES)] = (
            rows[j, e0 + t, pl.ds(LANES, LANES)] * sc)
      return _

    lax.fori_loop(0, SUB // LANES, body, None)

  def _dstu_copy(b, j):
    def body(g, _):
      dstu[j, pl.ds(g * LANES, LANES)] = dstb[b, j, pl.ds(g * LANES, LANES)]
      return _

    lax.fori_loop(0, SUB // LANES, body, None)

  def _stage(row, b, sem):
    pltpu.async_copy(gidx_all.at[row], idxb.at[b], sem)
    pltpu.async_copy(dst_all.at[row], dstb.at[b], sem)
    pltpu.async_copy(val_all.at[row], valb.at[b], sem)

  def _stage_wait(row, b, sem):
    pltpu.make_async_copy(gidx_all.at[row], idxb.at[b], sem).wait()
    pltpu.make_async_copy(dst_all.at[row], dstb.at[b], sem).wait()
    pltpu.make_async_copy(val_all.at[row], valb.at[b], sem).wait()

  zr0 = s * ROWS_PER_TILE
  nfull = ROWS_PER_TILE // SUB
  ztail = ROWS_PER_TILE - nfull * SUB
  iota16 = lax.iota(jnp.int32, LANES)

  def init_round(m, _):
    node0 = zr0 + m * SUB
    for g in range(SUB // LANES):
      node16 = jnp.minimum(node0 + g * LANES + iota16, N - 1)
      dstu[0, pl.ds(g * LANES, LANES)] = node16 * 4 + c
    pltpu.async_copy(hsv2.at[dstu.at[0]], rows.at[0], semG)
    pltpu.make_async_copy(hsv2.at[dstu.at[0]], rows.at[0], semG).wait()
    pltpu.sync_copy(rows.at[0], agg_sh.at[pl.ds(node0, SUB)])
    return _

  lax.fori_loop(0, nfull, init_round, None)
  node0t = zr0 + nfull * SUB
  for g in range(SUB // LANES):
    node16 = jnp.minimum(node0t + g * LANES + iota16, N - 1)
    dstu[0, pl.ds(g * LANES, LANES)] = node16 * 4 + c
  pltpu.async_copy(hsv2.at[dstu.at[0]], rows.at[0], semG)
  pltpu.make_async_copy(hsv2.at[dstu.at[0]], rows.at[0], semG).wait()
  pltpu.sync_copy(rows.at[0, pl.ds(0, ztail)],
                  agg_sh.at[pl.ds(node0t, ztail)])
  plsc.subcore_barrier()

  chunk0 = s * NCHUNKS

  pltpu.sync_copy(gidx_all.at[chunk0], idxb.at[0])
  pltpu.sync_copy(dst_all.at[chunk0], dstb.at[0])
  pltpu.sync_copy(val_all.at[chunk0], valb.at[0])
  pltpu.sync_copy(gidx_all.at[chunk0 + 1], idxb.at[1])
  pltpu.sync_copy(dst_all.at[chunk0 + 1], dstb.at[1])
  pltpu.sync_copy(val_all.at[chunk0 + 1], valb.at[1])
  _addc(0)
  _addc(1)
  pltpu.async_copy(hsv1.at[idxb.at[0, 0]], rows.at[0], semG)
  pltpu.async_copy(hsv1.at[idxb.at[0, 1]], rows.at[1], semG)

  def pair_body(p, _):
    for sb in range(2):
      lc = 2 * p + sb
      r = chunk0 + lc

      for j in range(KSUB):
        pltpu.make_async_copy(hsv1.at[idxb.at[sb, j]], rows.at[j],
                              semG).wait()
        j2 = (j - 2) % KSUB

        @pl.when(4 * lc + j >= 2)
        def _():
          pltpu.make_async_copy(rows.at[j2], agg_sh.at[dstu.at[j2]],
                                semC).wait()

        if j == 2:
          @pl.when(jnp.logical_and(lc >= 1, lc + 1 < NCHUNKS))
          def _():
            _stage_wait(r + 1, 1 - sb, semS[1 - sb])
            _addc(1 - sb)

        if j < 2:
          pltpu.async_copy(hsv1.at[idxb.at[sb, j + 2]], rows.at[j + 2], semG)
        else:
          @pl.when(lc + 1 < NCHUNKS)
          def _():
            pltpu.async_copy(hsv1.at[idxb.at[1 - sb, j - 2]], rows.at[j - 2],
                             semG)

        _dstu_copy(sb, j)
        _scale(sb, j)
        pltpu.async_copy(rows.at[j], agg_sh.at[dstu.at[j]], semC, add=True)

      @pl.when(lc + 2 < NCHUNKS)
      def _():
        _stage(r + 2, sb, semS[sb])
    return _

  lax.fori_loop(0, NCHUNKS // 2, pair_body, None)
  for j2 in (2, 3):
    pltpu.make_async_copy(rows.at[j2], agg_sh.at[dstu.at[j2]], semC).wait()
  plsc.subcore_barrier()

  t = s // 8
  srel = s - 8 * t
  outbase = (2 * t + c) * QP + srel * QW
  qc0 = s * QCH

  def _relu(j):
    def body(i, _):
      rows[j, i, pl.ds(0, LANES)] = jnp.maximum(
          rows[j, i, pl.ds(0, LANES)], 0.0)
      rows[j, i, pl.ds(LANES, LANES)] = jnp.maximum(
          rows[j, i, pl.ds(LANES, LANES)], 0.0)
      return _

    lax.fori_loop(0, SUB, body, None)

  pltpu.sync_copy(qidx.at[qc0], idxb.at[0])
  pltpu.sync_copy(qidx.at[qc0 + 1], idxb.at[1])
  pltpu.async_copy(agg_sh.at[idxb.at[0, 0]], rows.at[0], semG)
  pltpu.async_copy(agg_sh.at[idxb.at[0, 1]], rows.at[1], semG)

  def q_pair_body(p, _):
    for sb in range(2):
      lq = 2 * p + sb

      for j in range(KSUB):
        qu = 4 * lq + j
        pltpu.make_async_copy(agg_sh.at[idxb.at[sb, j]], rows.at[j],
                              semG).wait()
        j2 = (j - 2) % KSUB
        qu2 = qu - 2

        @pl.when(qu >= 2)
        def _():
          pltpu.make_async_copy(
              rows.at[j2],
              hp_out.at[pl.ds(outbase + qu2 * SUB, SUB)], semC).wait()

        if j == 2:
          @pl.when(jnp.logical_and(lq >= 1, lq + 1 < QCH))
          def _():
            pltpu.make_async_copy(qidx.at[qc0 + lq + 1], idxb.at[1 - sb],
                                  semS[1 - sb]).wait()

        if j < 2:
          pltpu.async_copy(agg_sh.at[idxb.at[sb, j + 2]], rows.at[j + 2],
                           semG)
        else:
          @pl.when(lq + 1 < QCH)
          def _():
            pltpu.async_copy(agg_sh.at[idxb.at[1 - sb, j - 2]],
                             rows.at[j - 2], semG)

        _relu(j)
        pltpu.async_copy(rows.at[j],
                         hp_out.at[pl.ds(outbase + qu * SUB, SUB)], semC)

      @pl.when(lq + 2 < QCH)
      def _():
        pltpu.async_copy(qidx.at[qc0 + lq + 2], idxb.at[sb], semS[sb])
    return _

  lax.fori_loop(0, QCH // 2, q_pair_body, None)
  for j2, qu2 in ((2, 4 * QCH - 2), (3, 4 * QCH - 1)):
    pltpu.make_async_copy(rows.at[j2],
                          hp_out.at[pl.ds(outbase + qu2 * SUB, SUB)],
                          semC).wait()


def _scatter(hsv1, hsv2, gidx_all, dst_all, val_all, qidx):
  k = pl.kernel(
      _scatter_body,
      out_type=jax.ShapeDtypeStruct((4 * QP, D // 2), jnp.float32),
      mesh=_mesh(),
      scratch_types=[
          pltpu.VMEM((2, KSUB, SUB), jnp.int32),
          pltpu.VMEM((2, KSUB, SUB), jnp.int32),
          pltpu.VMEM((2, KSUB, SUB), jnp.float32),
          pltpu.VMEM((KSUB, SUB), jnp.int32),
          pltpu.VMEM((KSUB, SUB, D // 2), jnp.float32),
          pltpu.VMEM_SHARED((NPAD, D // 2), jnp.float32),
          pltpu.SemaphoreType.DMA,
          pltpu.SemaphoreType.DMA,
          pltpu.SemaphoreType.DMA,
          pltpu.SemaphoreType.DMA,
      ],
      compiler_params=_SC_PARAMS,
  )
  return k(hsv1, hsv2, gidx_all, dst_all, val_all, qidx)


def _head_body(u0_ref, u1_ref, v0_ref, v1_ref, bu0_ref, bu1_ref, bv0_ref,
               bv1_ref, b1_ref, w2_ref, b2_ref, out_ref):
  z = jnp.maximum(
      jnp.dot(u0_ref[...], bu0_ref[...], preferred_element_type=jnp.float32)
      + jnp.dot(u1_ref[...], bu1_ref[...], preferred_element_type=jnp.float32)
      + jnp.dot(v0_ref[...], bv0_ref[...], preferred_element_type=jnp.float32)
      + jnp.dot(v1_ref[...], bv1_ref[...], preferred_element_type=jnp.float32)
      + b1_ref[...], 0.0)
  l = jnp.dot(z, w2_ref[...], preferred_element_type=jnp.float32) + b2_ref[...]
  parts = []
  for g in range(4):
    lg = l[:, 4 * g:4 * g + 4]
    m = jnp.max(lg, axis=-1, keepdims=True)
    e = jnp.exp(lg - m)
    parts.append(e / jnp.sum(e, axis=-1, keepdims=True))
  out_ref[...] = jnp.concatenate(parts, axis=-1)


BQ4 = 416
Q4 = Q // 4


def _head(hp4, bu0, bu1, bv0, bv1, b1, w2, b2):
  sec = QP // 4 // BQ4
  return pl.pallas_call(
      _head_body,
      grid=(-(-Q4 // BQ4),),
      in_specs=[
          pl.BlockSpec((BQ4, 2 * D), lambda i: (i, 0)),
          pl.BlockSpec((BQ4, 2 * D), lambda i: (sec + i, 0)),
          pl.BlockSpec((BQ4, 2 * D), lambda i: (2 * sec + i, 0)),
          pl.BlockSpec((BQ4, 2 * D), lambda i: (3 * sec + i, 0)),
          pl.BlockSpec((2 * D, 4 * D), lambda i: (0, 0)),
          pl.BlockSpec((2 * D, 4 * D), lambda i: (0, 0)),
          pl.BlockSpec((2 * D, 4 * D), lambda i: (0, 0)),
          pl.BlockSpec((2 * D, 4 * D), lambda i: (0, 0)),
          pl.BlockSpec((1, 4 * D), lambda i: (0, 0)),
          pl.BlockSpec((4 * D, 16), lambda i: (0, 0)),
          pl.BlockSpec((1, 16), lambda i: (0, 0)),
      ],
      out_specs=pl.BlockSpec((BQ4, 16), lambda i: (i, 0)),
      out_shape=jax.ShapeDtypeStruct((Q4, 16), jnp.float32),
  )(hp4, hp4, hp4, hp4, bu0, bu1, bv0, bv1, b1, w2, b2)


def kernel(node_ids, omni_idx, omni_val, string_idx, string_val, query_edges,
           emb_table, W_self, W_omni, W_string, Wc1, bc1, Wc2, bc2):
  wcat1 = jnp.concatenate([W_omni, W_string], axis=1)
  wcat2 = jnp.concatenate([W_self, W_self], axis=1)
  hs1, hs2 = _proj(emb_table, wcat1, wcat2)
  hsv1 = hs1.reshape(4 * N, D // 2)
  hsv2 = hs2.reshape(4 * N, D // 2)

  pad = EEP - EE
  par = jnp.arange(pad, dtype=jnp.int32)
  gidx_all = jnp.concatenate([
      omni_idx[:, 1] * 4,
      string_idx[:, 1] * 4 + 2,
      (par % N) * 4,
  ]).reshape(EEP // CHUNK, KSUB, SUB)
  dst_all = jnp.concatenate(
      [omni_idx[:, 0], string_idx[:, 0],
       par % N]).reshape(EEP // CHUNK, KSUB, SUB)
  val_all = jnp.concatenate([
      omni_val, string_val, jnp.zeros((pad,), jnp.float32)
  ]).reshape(EEP // CHUNK, KSUB, SUB)

  qpad = QP - Q
  qpar = jnp.arange(qpad, dtype=jnp.int32) % N
  qidx = jnp.concatenate([
      query_edges[:, 0], qpar,
      query_edges[:, 1], qpar,
  ]).reshape(NC * QP // CHUNK, KSUB, SUB)

  hp = _scatter(hsv1, hsv2, gidx_all, dst_all, val_all,
                qidx)
  hp4 = hp.reshape(QP, 2 * D)

  eye4 = jnp.eye(4, dtype=jnp.float32)
  w1a, w1b = Wc1[:D], Wc1[D:]
  bu0 = jnp.kron(eye4, w1a[:D // 2])
  bu1 = jnp.kron(eye4, w1a[D // 2:])
  bv0 = jnp.kron(eye4, w1b[:D // 2])
  bv1 = jnp.kron(eye4, w1b[D // 2:])
  b1 = jnp.tile(bc1, 4).reshape(1, 4 * D)
  w2 = jnp.kron(eye4, Wc2)
  b2 = jnp.tile(bc2, 4).reshape(1, 16)

  probs4 = _head(hp4, bu0, bu1, bv0, bv1, b1, w2, b2)
  return probs4.reshape(Q, 4)


# speedup vs baseline: 1.0301x; 1.0023x over previous
import functools

import jax
import jax.numpy as jnp
from jax import lax
from jax.experimental import pallas as pl
from jax.experimental.pallas import tpu as pltpu
from jax.experimental.pallas import tpu_sc as plsc

N = 50000
E = 800000
D = 64
Q = 100000

NC = 2
NS = 16
LANES = 16

EE = 2 * E
SUB = 128
KSUB = 4
CHUNK = SUB * KSUB
EPW_RAW = -(-EE // NS)
EPW = -(-EPW_RAW // CHUNK) * CHUNK
EEP = EPW * NS
NCHUNKS = EPW // CHUNK

NPAD = 50048
ROWS_PER_TILE = NPAD // NS

QPW = -(-Q // (NS * CHUNK)) * CHUNK
QP = QPW * NS


def _mesh():
  return plsc.VectorSubcoreMesh(core_axis_name="c", subcore_axis_name="s")


_SC_PARAMS = pltpu.CompilerParams(use_tc_tiling_on_sc=False)


def _proj_body(emb_ref, w1_ref, w2_ref, out1_ref, out2_ref):
  out1_ref[...] = jnp.dot(emb_ref[...], w1_ref[...],
                          preferred_element_type=jnp.float32)
  out2_ref[...] = jnp.dot(emb_ref[...], w2_ref[...],
                          preferred_element_type=jnp.float32)


def _proj(emb, wcat1, wcat2):
  bn = 1000
  return pl.pallas_call(
      _proj_body,
      grid=(N // bn,),
      in_specs=[
          pl.BlockSpec((bn, D), lambda i: (i, 0)),
          pl.BlockSpec((D, 2 * D), lambda i: (0, 0)),
          pl.BlockSpec((D, 2 * D), lambda i: (0, 0)),
      ],
      out_specs=[
          pl.BlockSpec((bn, 2 * D), lambda i: (i, 0)),
          pl.BlockSpec((bn, 2 * D), lambda i: (i, 0)),
      ],
      out_shape=[
          jax.ShapeDtypeStruct((N, 2 * D), jnp.float32),
          jax.ShapeDtypeStruct((N, 2 * D), jnp.float32),
      ],
  )(emb, wcat1, wcat2)


QW = 2 * QP // NS
QUNITS = QW // SUB
QCH = QW // CHUNK


def _scatter_body(hsv1, hsv2, gidx_all, dst_all, val_all, qidx, hp_out,
                  idxb, dstb, valb, dstu, rows, agg_sh,
                  semS0, semS1, semG, semC):
  c = lax.axis_index("c")
  s = lax.axis_index("s")
  semS = (semS0, semS1)

  def _addc(b):
    for j in range(KSUB):
      def body(g, _, j=j):
        idxb[b, j, pl.ds(g * LANES, LANES)] = (
            idxb[b, j, pl.ds(g * LANES, LANES)] + c)
        return _

      lax.fori_loop(0, SUB // LANES, body, None)

  def _scale(b, j):
    def body(g, _):
      v16 = valb[b, j, pl.ds(g * LANES, LANES)]
      e0 = g * LANES
      for t in range(LANES):
        sc = v16[t]
        rows[j, e0 + t, pl.ds(0, LANES)] = (
            rows[j, e0 + t, pl.ds(0, LANES)] * sc)
        rows[j, e0 + t, pl.ds(LANES, LANES)] = (
            rows[j, e0 + t, pl.ds(LANES, LANES)] * sc)
      return _

    lax.fori_loop(0, SUB // LANES, body, None)

  def _dstu_copy(b, j):
    def body(g, _):
      dstu[j, pl.ds(g * LANES, LANES)] = dstb[b, j, pl.ds(g * LANES, LANES)]
      return _

    lax.fori_loop(0, SUB // LANES, body, None)

  def _stage(row, b, sem):
    pltpu.async_copy(gidx_all.at[row], idxb.at[b], sem)
    pltpu.async_copy(dst_all.at[row], dstb.at[b], sem)
    pltpu.async_copy(val_all.at[row], valb.at[b], sem)

  def _stage_wait(row, b, sem):
    pltpu.make_async_copy(gidx_all.at[row], idxb.at[b], sem).wait()
    pltpu.make_async_copy(dst_all.at[row], dstb.at[b], sem).wait()
    pltpu.make_async_copy(val_all.at[row], valb.at[b], sem).wait()

  zr0 = s * ROWS_PER_TILE
  nfull = ROWS_PER_TILE // SUB
  ztail = ROWS_PER_TILE - nfull * SUB
  iota16 = lax.iota(jnp.int32, LANES)

  def init_round(m, _):
    node0 = zr0 + m * SUB
    for g in range(SUB // LANES):
      node16 = jnp.minimum(node0 + g * LANES + iota16, N - 1)
      dstu[0, pl.ds(g * LANES, LANES)] = node16 * 4 + c
    pltpu.async_copy(hsv2.at[dstu.at[0]], rows.at[0], semG)
    pltpu.make_async_copy(hsv2.at[dstu.at[0]], rows.at[0], semG).wait()
    pltpu.sync_copy(rows.at[0], agg_sh.at[pl.ds(node0, SUB)])
    return _

  lax.fori_loop(0, nfull, init_round, None)
  node0t = zr0 + nfull * SUB
  for g in range(SUB // LANES):
    node16 = jnp.minimum(node0t + g * LANES + iota16, N - 1)
    dstu[0, pl.ds(g * LANES, LANES)] = node16 * 4 + c
  pltpu.async_copy(hsv2.at[dstu.at[0]], rows.at[0], semG)
  pltpu.make_async_copy(hsv2.at[dstu.at[0]], rows.at[0], semG).wait()
  pltpu.sync_copy(rows.at[0, pl.ds(0, ztail)],
                  agg_sh.at[pl.ds(node0t, ztail)])
  plsc.subcore_barrier()

  chunk0 = s * NCHUNKS

  pltpu.sync_copy(gidx_all.at[chunk0], idxb.at[0])
  pltpu.sync_copy(dst_all.at[chunk0], dstb.at[0])
  pltpu.sync_copy(val_all.at[chunk0], valb.at[0])
  pltpu.sync_copy(gidx_all.at[chunk0 + 1], idxb.at[1])
  pltpu.sync_copy(dst_all.at[chunk0 + 1], dstb.at[1])
  pltpu.sync_copy(val_all.at[chunk0 + 1], valb.at[1])
  _addc(0)
  _addc(1)
  pltpu.async_copy(hsv1.at[idxb.at[0, 0]], rows.at[0], semG)
  pltpu.async_copy(hsv1.at[idxb.at[0, 1]], rows.at[1], semG)

  def pair_body(p, _):
    for sb in range(2):
      lc = 2 * p + sb
      r = chunk0 + lc

      for j in range(KSUB):
        pltpu.make_async_copy(hsv1.at[idxb.at[sb, j]], rows.at[j],
                              semG).wait()
        j2 = (j - 2) % KSUB

        @pl.when(4 * lc + j >= 2)
        def _():
          pltpu.make_async_copy(rows.at[j2], agg_sh.at[dstu.at[j2]],
                                semC).wait()

        if j == 2:
          @pl.when(jnp.logical_and(lc >= 1, lc + 1 < NCHUNKS))
          def _():
            _stage_wait(r + 1, 1 - sb, semS[1 - sb])
            _addc(1 - sb)

        if j < 2:
          pltpu.async_copy(hsv1.at[idxb.at[sb, j + 2]], rows.at[j + 2], semG)
        else:
          @pl.when(lc + 1 < NCHUNKS)
          def _():
            pltpu.async_copy(hsv1.at[idxb.at[1 - sb, j - 2]], rows.at[j - 2],
                             semG)

        _dstu_copy(sb, j)
        _scale(sb, j)
        pltpu.async_copy(rows.at[j], agg_sh.at[dstu.at[j]], semC, add=True)

      @pl.when(lc + 2 < NCHUNKS)
      def _():
        _stage(r + 2, sb, semS[sb])
    return _

  lax.fori_loop(0, NCHUNKS // 2, pair_body, None)
  for j2 in (2, 3):
    pltpu.make_async_copy(rows.at[j2], agg_sh.at[dstu.at[j2]], semC).wait()
  plsc.subcore_barrier()

  t = s // 8
  srel = s - 8 * t
  outbase = (2 * t + c) * QP + srel * QW
  qc0 = s * QCH

  def _relu(j):
    def body(i, _):
      rows[j, i, pl.ds(0, LANES)] = jnp.maximum(
          rows[j, i, pl.ds(0, LANES)], 0.0)
      rows[j, i, pl.ds(LANES, LANES)] = jnp.maximum(
          rows[j, i, pl.ds(LANES, LANES)], 0.0)
      return _

    lax.fori_loop(0, SUB, body, None)

  pltpu.sync_copy(qidx.at[qc0], idxb.at[0])
  pltpu.sync_copy(qidx.at[qc0 + 1], idxb.at[1])
  pltpu.async_copy(agg_sh.at[idxb.at[0, 0]], rows.at[0], semG)
  pltpu.async_copy(agg_sh.at[idxb.at[0, 1]], rows.at[1], semG)

  def q_pair_body(p, _):
    for sb in range(2):
      lq = 2 * p + sb

      for j in range(KSUB):
        qu = 4 * lq + j
        pltpu.make_async_copy(agg_sh.at[idxb.at[sb, j]], rows.at[j],
                              semG).wait()
        j2 = (j - 2) % KSUB
        qu2 = qu - 2

        @pl.when(qu >= 2)
        def _():
          pltpu.make_async_copy(
              rows.at[j2],
              hp_out.at[pl.ds(outbase + qu2 * SUB, SUB)], semC).wait()

        if j == 2:
          @pl.when(jnp.logical_and(lq >= 1, lq + 1 < QCH))
          def _():
            pltpu.make_async_copy(qidx.at[qc0 + lq + 1], idxb.at[1 - sb],
                                  semS[1 - sb]).wait()

        if j < 2:
          pltpu.async_copy(agg_sh.at[idxb.at[sb, j + 2]], rows.at[j + 2],
                           semG)
        else:
          @pl.when(lq + 1 < QCH)
          def _():
            pltpu.async_copy(agg_sh.at[idxb.at[1 - sb, j - 2]],
                             rows.at[j - 2], semG)

        _relu(j)
        pltpu.async_copy(rows.at[j],
                         hp_out.at[pl.ds(outbase + qu * SUB, SUB)], semC)

      @pl.when(lq + 2 < QCH)
      def _():
        pltpu.async_copy(qidx.at[qc0 + lq + 2], idxb.at[sb], semS[sb])
    return _

  lax.fori_loop(0, QCH // 2, q_pair_body, None)
  for j2, qu2 in ((2, 4 * QCH - 2), (3, 4 * QCH - 1)):
    pltpu.make_async_copy(rows.at[j2],
                          hp_out.at[pl.ds(outbase + qu2 * SUB, SUB)],
                          semC).wait()


def _scatter(hsv1, hsv2, gidx_all, dst_all, val_all, qidx):
  k = pl.kernel(
      _scatter_body,
      out_type=jax.ShapeDtypeStruct((4 * QP, D // 2), jnp.float32),
      mesh=_mesh(),
      scratch_types=[
          pltpu.VMEM((2, KSUB, SUB), jnp.int32),
          pltpu.VMEM((2, KSUB, SUB), jnp.int32),
          pltpu.VMEM((2, KSUB, SUB), jnp.float32),
          pltpu.VMEM((KSUB, SUB), jnp.int32),
          pltpu.VMEM((KSUB, SUB, D // 2), jnp.float32),
          pltpu.VMEM_SHARED((NPAD, D // 2), jnp.float32),
          pltpu.SemaphoreType.DMA,
          pltpu.SemaphoreType.DMA,
          pltpu.SemaphoreType.DMA,
          pltpu.SemaphoreType.DMA,
      ],
      compiler_params=_SC_PARAMS,
  )
  return k(hsv1, hsv2, gidx_all, dst_all, val_all, qidx)


def _head_body(hp_ref, bu0_ref, bu1_ref, bv0_ref, bv1_ref, b1_ref, w2_ref,
               b2_ref, out_ref):
  z = jnp.maximum(
      jnp.dot(hp_ref[0], bu0_ref[...], preferred_element_type=jnp.float32)
      + jnp.dot(hp_ref[1], bu1_ref[...], preferred_element_type=jnp.float32)
      + jnp.dot(hp_ref[2], bv0_ref[...], preferred_element_type=jnp.float32)
      + jnp.dot(hp_ref[3], bv1_ref[...], preferred_element_type=jnp.float32)
      + b1_ref[...], 0.0)
  l = jnp.dot(z, w2_ref[...], preferred_element_type=jnp.float32) + b2_ref[...]
  parts = []
  for g in range(4):
    lg = l[:, 4 * g:4 * g + 4]
    m = jnp.max(lg, axis=-1, keepdims=True)
    e = jnp.exp(lg - m)
    parts.append(e / jnp.sum(e, axis=-1, keepdims=True))
  out_ref[...] = jnp.concatenate(parts, axis=-1)


BQ4 = 416
Q4 = Q // 4


def _head(hp3, bu0, bu1, bv0, bv1, b1, w2, b2):
  return pl.pallas_call(
      _head_body,
      grid=(-(-Q4 // BQ4),),
      in_specs=[
          pl.BlockSpec((4, BQ4, 2 * D), lambda i: (0, i, 0)),
          pl.BlockSpec((2 * D, 4 * D), lambda i: (0, 0)),
          pl.BlockSpec((2 * D, 4 * D), lambda i: (0, 0)),
          pl.BlockSpec((2 * D, 4 * D), lambda i: (0, 0)),
          pl.BlockSpec((2 * D, 4 * D), lambda i: (0, 0)),
          pl.BlockSpec((1, 4 * D), lambda i: (0, 0)),
          pl.BlockSpec((4 * D, 16), lambda i: (0, 0)),
          pl.BlockSpec((1, 16), lambda i: (0, 0)),
      ],
      out_specs=pl.BlockSpec((BQ4, 16), lambda i: (i, 0)),
      out_shape=jax.ShapeDtypeStruct((Q4, 16), jnp.float32),
  )(hp3, bu0, bu1, bv0, bv1, b1, w2, b2)


def kernel(node_ids, omni_idx, omni_val, string_idx, string_val, query_edges,
           emb_table, W_self, W_omni, W_string, Wc1, bc1, Wc2, bc2):
  wcat1 = jnp.concatenate([W_omni, W_string], axis=1)
  wcat2 = jnp.concatenate([W_self, W_self], axis=1)
  hs1, hs2 = _proj(emb_table, wcat1, wcat2)
  hsv1 = hs1.reshape(4 * N, D // 2)
  hsv2 = hs2.reshape(4 * N, D // 2)

  pad = EEP - EE
  par = jnp.arange(pad, dtype=jnp.int32)
  gidx_all = jnp.concatenate([
      omni_idx[:, 1] * 4,
      string_idx[:, 1] * 4 + 2,
      (par % N) * 4,
  ]).reshape(EEP // CHUNK, KSUB, SUB)
  dst_all = jnp.concatenate(
      [omni_idx[:, 0], string_idx[:, 0],
       par % N]).reshape(EEP // CHUNK, KSUB, SUB)
  val_all = jnp.concatenate([
      omni_val, string_val, jnp.zeros((pad,), jnp.float32)
  ]).reshape(EEP // CHUNK, KSUB, SUB)

  qpad = QP - Q
  qpar = jnp.arange(qpad, dtype=jnp.int32) % N
  qidx = jnp.concatenate([
      query_edges[:, 0], qpar,
      query_edges[:, 1], qpar,
  ]).reshape(NC * QP // CHUNK, KSUB, SUB)

  hp = _scatter(hsv1, hsv2, gidx_all, dst_all, val_all,
                qidx)
  hp3 = hp.reshape(4, QP // 4, 2 * D)

  eye4 = jnp.eye(4, dtype=jnp.float32)
  w1a, w1b = Wc1[:D], Wc1[D:]
  bu0 = jnp.kron(eye4, w1a[:D // 2])
  bu1 = jnp.kron(eye4, w1a[D // 2:])
  bv0 = jnp.kron(eye4, w1b[:D // 2])
  bv1 = jnp.kron(eye4, w1b[D // 2:])
  b1 = jnp.tile(bc1, 4).reshape(1, 4 * D)
  w2 = jnp.kron(eye4, Wc2)
  b2 = jnp.tile(bc2, 4).reshape(1, 16)

  probs4 = _head(hp3, bu0, bu1, bv0, bv1, b1, w2, b2)
  return probs4.reshape(Q, 4)


# speedup vs baseline: 1.1165x; 1.0839x over previous
import functools

import jax
import jax.numpy as jnp
from jax import lax
from jax.experimental import pallas as pl
from jax.experimental.pallas import tpu as pltpu
from jax.experimental.pallas import tpu_sc as plsc

N = 50000
E = 800000
D = 64
Q = 100000

NC = 2
NS = 16
LANES = 16

EE = 2 * E
SUB = 128
KSUB = 4
CHUNK = SUB * KSUB
EPW_RAW = -(-EE // NS)
EPW = -(-EPW_RAW // CHUNK) * CHUNK
EEP = EPW * NS
NCHUNKS = EPW // CHUNK

NPAD = 50048
ROWS_PER_TILE = NPAD // NS

QPW = -(-Q // (NS * CHUNK)) * CHUNK
QP = QPW * NS


def _mesh():
  return plsc.VectorSubcoreMesh(core_axis_name="c", subcore_axis_name="s")


_SC_PARAMS = pltpu.CompilerParams(use_tc_tiling_on_sc=False)


def _proj_body(emb_ref, w1_ref, w2_ref, out1_ref, out2_ref):
  out1_ref[...] = jnp.dot(emb_ref[...], w1_ref[...],
                          preferred_element_type=jnp.float32)
  out2_ref[...] = jnp.dot(emb_ref[...], w2_ref[...],
                          preferred_element_type=jnp.float32)


def _proj(emb, wcat1, wcat2):
  bn = 2000
  return pl.pallas_call(
      _proj_body,
      grid=(N // bn,),
      in_specs=[
          pl.BlockSpec((bn, D), lambda i: (i, 0)),
          pl.BlockSpec((D, 2 * D), lambda i: (0, 0)),
          pl.BlockSpec((D, 2 * D), lambda i: (0, 0)),
      ],
      out_specs=[
          pl.BlockSpec((bn, 2 * D), lambda i: (i, 0)),
          pl.BlockSpec((bn, 2 * D), lambda i: (i, 0)),
      ],
      out_shape=[
          jax.ShapeDtypeStruct((N, 2 * D), jnp.float32),
          jax.ShapeDtypeStruct((N, 2 * D), jnp.float32),
      ],
  )(emb, wcat1, wcat2)


QW = 2 * QP // NS
QUNITS = QW // SUB
QCH = QW // CHUNK


def _scatter_body(hsv1, hsv2, gidx_all, dst_all, val_all, qidx, hp_out,
                  idxb, dstb, valb, dstu, rows, agg_sh,
                  semS0, semS1, semG, semC):
  c = lax.axis_index("c")
  s = lax.axis_index("s")
  semS = (semS0, semS1)

  def _addc(b):
    for j in range(KSUB):
      def body(g, _, j=j):
        idxb[b, j, pl.ds(g * LANES, LANES)] = (
            idxb[b, j, pl.ds(g * LANES, LANES)] + c)
        return _

      lax.fori_loop(0, SUB // LANES, body, None)

  def _scale(b, j):
    def body(g, _):
      v16 = valb[b, j, pl.ds(g * LANES, LANES)]
      e0 = g * LANES
      for t in range(LANES):
        sc = v16[t]
        rows[j, e0 + t, pl.ds(0, LANES)] = (
            rows[j, e0 + t, pl.ds(0, LANES)] * sc)
        rows[j, e0 + t, pl.ds(LANES, LANES)] = (
            rows[j, e0 + t, pl.ds(LANES, LANES)] * sc)
      return _

    lax.fori_loop(0, SUB // LANES, body, None)

  def _dstu_copy(b, j):
    def body(g, _):
      dstu[j, pl.ds(g * LANES, LANES)] = dstb[b, j, pl.ds(g * LANES, LANES)]
      return _

    lax.fori_loop(0, SUB // LANES, body, None)

  def _stage(row, b, sem):
    pltpu.async_copy(gidx_all.at[row], idxb.at[b], sem)
    pltpu.async_copy(dst_all.at[row], dstb.at[b], sem)
    pltpu.async_copy(val_all.at[row], valb.at[b], sem)

  def _stage_wait(row, b, sem):
    pltpu.make_async_copy(gidx_all.at[row], idxb.at[b], sem).wait()
    pltpu.make_async_copy(dst_all.at[row], dstb.at[b], sem).wait()
    pltpu.make_async_copy(val_all.at[row], valb.at[b], sem).wait()

  zr0 = s * ROWS_PER_TILE
  nfull = ROWS_PER_TILE // SUB
  ztail = ROWS_PER_TILE - nfull * SUB
  iota16 = lax.iota(jnp.int32, LANES)

  def init_round(m, _):
    node0 = zr0 + m * SUB
    for g in range(SUB // LANES):
      node16 = jnp.minimum(node0 + g * LANES + iota16, N - 1)
      dstu[0, pl.ds(g * LANES, LANES)] = node16 * 4 + c
    pltpu.async_copy(hsv2.at[dstu.at[0]], rows.at[0], semG)
    pltpu.make_async_copy(hsv2.at[dstu.at[0]], rows.at[0], semG).wait()
    pltpu.sync_copy(rows.at[0], agg_sh.at[pl.ds(node0, SUB)])
    return _

  lax.fori_loop(0, nfull, init_round, None)
  node0t = zr0 + nfull * SUB
  for g in range(SUB // LANES):
    node16 = jnp.minimum(node0t + g * LANES + iota16, N - 1)
    dstu[0, pl.ds(g * LANES, LANES)] = node16 * 4 + c
  pltpu.async_copy(hsv2.at[dstu.at[0]], rows.at[0], semG)
  pltpu.make_async_copy(hsv2.at[dstu.at[0]], rows.at[0], semG).wait()
  pltpu.sync_copy(rows.at[0, pl.ds(0, ztail)],
                  agg_sh.at[pl.ds(node0t, ztail)])
  plsc.subcore_barrier()

  chunk0 = s * NCHUNKS

  pltpu.sync_copy(gidx_all.at[chunk0], idxb.at[0])
  pltpu.sync_copy(dst_all.at[chunk0], dstb.at[0])
  pltpu.sync_copy(val_all.at[chunk0], valb.at[0])
  pltpu.sync_copy(gidx_all.at[chunk0 + 1], idxb.at[1])
  pltpu.sync_copy(dst_all.at[chunk0 + 1], dstb.at[1])
  pltpu.sync_copy(val_all.at[chunk0 + 1], valb.at[1])
  _addc(0)
  _addc(1)
  pltpu.async_copy(hsv1.at[idxb.at[0, 0]], rows.at[0], semG)
  pltpu.async_copy(hsv1.at[idxb.at[0, 1]], rows.at[1], semG)

  def pair_body(p, _):
    for sb in range(2):
      lc = 2 * p + sb
      r = chunk0 + lc

      for j in range(KSUB):
        pltpu.make_async_copy(hsv1.at[idxb.at[sb, j]], rows.at[j],
                              semG).wait()
        j2 = (j - 2) % KSUB

        @pl.when(4 * lc + j >= 2)
        def _():
          pltpu.make_async_copy(rows.at[j2], agg_sh.at[dstu.at[j2]],
                                semC).wait()

        if j == 2:
          @pl.when(jnp.logical_and(lc >= 1, lc + 1 < NCHUNKS))
          def _():
            _stage_wait(r + 1, 1 - sb, semS[1 - sb])
            _addc(1 - sb)

        if j < 2:
          pltpu.async_copy(hsv1.at[idxb.at[sb, j + 2]], rows.at[j + 2], semG)
        else:
          @pl.when(lc + 1 < NCHUNKS)
          def _():
            pltpu.async_copy(hsv1.at[idxb.at[1 - sb, j - 2]], rows.at[j - 2],
                             semG)

        _dstu_copy(sb, j)
        _scale(sb, j)
        pltpu.async_copy(rows.at[j], agg_sh.at[dstu.at[j]], semC, add=True)

      @pl.when(lc + 2 < NCHUNKS)
      def _():
        _stage(r + 2, sb, semS[sb])
    return _

  lax.fori_loop(0, NCHUNKS // 2, pair_body, None)
  for j2 in (2, 3):
    pltpu.make_async_copy(rows.at[j2], agg_sh.at[dstu.at[j2]], semC).wait()
  plsc.subcore_barrier()

  t = s // 8
  srel = s - 8 * t
  outbase = (2 * t + c) * QP + srel * QW
  qc0 = s * QCH

  def _relu(j):
    def body(i, _):
      rows[j, i, pl.ds(0, LANES)] = jnp.maximum(
          rows[j, i, pl.ds(0, LANES)], 0.0)
      rows[j, i, pl.ds(LANES, LANES)] = jnp.maximum(
          rows[j, i, pl.ds(LANES, LANES)], 0.0)
      return _

    lax.fori_loop(0, SUB, body, None)

  pltpu.sync_copy(qidx.at[qc0], idxb.at[0])
  pltpu.sync_copy(qidx.at[qc0 + 1], idxb.at[1])
  pltpu.async_copy(agg_sh.at[idxb.at[0, 0]], rows.at[0], semG)
  pltpu.async_copy(agg_sh.at[idxb.at[0, 1]], rows.at[1], semG)

  def q_pair_body(p, _):
    for sb in range(2):
      lq = 2 * p + sb

      for j in range(KSUB):
        qu = 4 * lq + j
        pltpu.make_async_copy(agg_sh.at[idxb.at[sb, j]], rows.at[j],
                              semG).wait()
        j2 = (j - 2) % KSUB
        qu2 = qu - 2

        @pl.when(qu >= 2)
        def _():
          pltpu.make_async_copy(
              rows.at[j2],
              hp_out.at[pl.ds(outbase + qu2 * SUB, SUB)], semC).wait()

        if j == 2:
          @pl.when(jnp.logical_and(lq >= 1, lq + 1 < QCH))
          def _():
            pltpu.make_async_copy(qidx.at[qc0 + lq + 1], idxb.at[1 - sb],
                                  semS[1 - sb]).wait()

        if j < 2:
          pltpu.async_copy(agg_sh.at[idxb.at[sb, j + 2]], rows.at[j + 2],
                           semG)
        else:
          @pl.when(lq + 1 < QCH)
          def _():
            pltpu.async_copy(agg_sh.at[idxb.at[1 - sb, j - 2]],
                             rows.at[j - 2], semG)

        _relu(j)
        pltpu.async_copy(rows.at[j],
                         hp_out.at[pl.ds(outbase + qu * SUB, SUB)], semC)

      @pl.when(lq + 2 < QCH)
      def _():
        pltpu.async_copy(qidx.at[qc0 + lq + 2], idxb.at[sb], semS[sb])
    return _

  lax.fori_loop(0, QCH // 2, q_pair_body, None)
  for j2, qu2 in ((2, 4 * QCH - 2), (3, 4 * QCH - 1)):
    pltpu.make_async_copy(rows.at[j2],
                          hp_out.at[pl.ds(outbase + qu2 * SUB, SUB)],
                          semC).wait()


def _scatter(hsv1, hsv2, gidx_all, dst_all, val_all, qidx):
  k = pl.kernel(
      _scatter_body,
      out_type=jax.ShapeDtypeStruct((4 * QP, D // 2), jnp.float32),
      mesh=_mesh(),
      scratch_types=[
          pltpu.VMEM((2, KSUB, SUB), jnp.int32),
          pltpu.VMEM((2, KSUB, SUB), jnp.int32),
          pltpu.VMEM((2, KSUB, SUB), jnp.float32),
          pltpu.VMEM((KSUB, SUB), jnp.int32),
          pltpu.VMEM((KSUB, SUB, D // 2), jnp.float32),
          pltpu.VMEM_SHARED((NPAD, D // 2), jnp.float32),
          pltpu.SemaphoreType.DMA,
          pltpu.SemaphoreType.DMA,
          pltpu.SemaphoreType.DMA,
          pltpu.SemaphoreType.DMA,
      ],
      compiler_params=_SC_PARAMS,
  )
  return k(hsv1, hsv2, gidx_all, dst_all, val_all, qidx)


def _head_body(hp_ref, bu0_ref, bu1_ref, bv0_ref, bv1_ref, b1_ref, w2_ref,
               b2_ref, gsum_ref, out_ref):
  z = jnp.maximum(
      jnp.dot(hp_ref[0], bu0_ref[...], preferred_element_type=jnp.float32)
      + jnp.dot(hp_ref[1], bu1_ref[...], preferred_element_type=jnp.float32)
      + jnp.dot(hp_ref[2], bv0_ref[...], preferred_element_type=jnp.float32)
      + jnp.dot(hp_ref[3], bv1_ref[...], preferred_element_type=jnp.float32)
      + b1_ref[...], 0.0)
  l = jnp.dot(z, w2_ref[...], preferred_element_type=jnp.float32) + b2_ref[...]
  e = jnp.exp(l - jnp.max(l, axis=-1, keepdims=True))
  s = jnp.dot(e, gsum_ref[...], preferred_element_type=jnp.float32)
  out_ref[...] = e / s


BQ4 = 416
Q4 = Q // 4


def _head(hp3, bu0, bu1, bv0, bv1, b1, w2, b2, gsum):
  return pl.pallas_call(
      _head_body,
      grid=(-(-Q4 // BQ4),),
      in_specs=[
          pl.BlockSpec((4, BQ4, 2 * D), lambda i: (0, i, 0)),
          pl.BlockSpec((2 * D, 4 * D), lambda i: (0, 0)),
          pl.BlockSpec((2 * D, 4 * D), lambda i: (0, 0)),
          pl.BlockSpec((2 * D, 4 * D), lambda i: (0, 0)),
          pl.BlockSpec((2 * D, 4 * D), lambda i: (0, 0)),
          pl.BlockSpec((1, 4 * D), lambda i: (0, 0)),
          pl.BlockSpec((4 * D, 16), lambda i: (0, 0)),
          pl.BlockSpec((1, 16), lambda i: (0, 0)),
          pl.BlockSpec((16, 16), lambda i: (0, 0)),
      ],
      out_specs=pl.BlockSpec((BQ4, 16), lambda i: (i, 0)),
      out_shape=jax.ShapeDtypeStruct((Q4, 16), jnp.float32),
  )(hp3, bu0, bu1, bv0, bv1, b1, w2, b2, gsum)


def kernel(node_ids, omni_idx, omni_val, string_idx, string_val, query_edges,
           emb_table, W_self, W_omni, W_string, Wc1, bc1, Wc2, bc2):
  wcat1 = jnp.concatenate([W_omni, W_string], axis=1)
  wcat2 = jnp.concatenate([W_self, W_self], axis=1)
  hs1, hs2 = _proj(emb_table, wcat1, wcat2)
  hsv1 = hs1.reshape(4 * N, D // 2)
  hsv2 = hs2.reshape(4 * N, D // 2)

  pad = EEP - EE
  par = jnp.arange(pad, dtype=jnp.int32)
  gidx_all = jnp.concatenate([
      omni_idx[:, 1] * 4,
      string_idx[:, 1] * 4 + 2,
      (par % N) * 4,
  ]).reshape(EEP // CHUNK, KSUB, SUB)
  dst_all = jnp.concatenate(
      [omni_idx[:, 0], string_idx[:, 0],
       par % N]).reshape(EEP // CHUNK, KSUB, SUB)
  val_all = jnp.concatenate([
      omni_val, string_val, jnp.zeros((pad,), jnp.float32)
  ]).reshape(EEP // CHUNK, KSUB, SUB)

  qpad = QP - Q
  qpar = jnp.arange(qpad, dtype=jnp.int32) % N
  qidx = jnp.concatenate([
      query_edges[:, 0], qpar,
      query_edges[:, 1], qpar,
  ]).reshape(NC * QP // CHUNK, KSUB, SUB)

  hp = _scatter(hsv1, hsv2, gidx_all, dst_all, val_all,
                qidx)
  hp3 = hp.reshape(4, QP // 4, 2 * D)

  eye4 = jnp.eye(4, dtype=jnp.float32)
  w1a, w1b = Wc1[:D], Wc1[D:]
  bu0 = jnp.kron(eye4, w1a[:D // 2])
  bu1 = jnp.kron(eye4, w1a[D // 2:])
  bv0 = jnp.kron(eye4, w1b[:D // 2])
  bv1 = jnp.kron(eye4, w1b[D // 2:])
  b1 = jnp.tile(bc1, 4).reshape(1, 4 * D)
  w2 = jnp.kron(eye4, Wc2)
  b2 = jnp.tile(bc2, 4).reshape(1, 16)

  gsum = jnp.kron(eye4, jnp.ones((4, 4), jnp.float32))
  probs4 = _head(hp3, bu0, bu1, bv0, bv1, b1, w2, b2, gsum)
  return probs4.reshape(Q, 4)
